# baseline (speedup 1.0000x reference)
import jax
import jax.numpy as jnp
from jax.experimental import pallas as pl
from jax.experimental.pallas import tpu as pltpu

_DIM = 64
_EPS = 1e-8
_FINAL_SCALE = _DIM ** -0.5
_NLEV = 256

_K_LUT = 256
_LUT_LO = 0.0
_LUT_STEP = 5.12 / _K_LUT
_LUT_INV_STEP = 1.0 / _LUT_STEP
_V_CLIP = 5.1
_BIG = 1e30

_PACKED_ROWS_PER_BLOCK = 1024


def _lut_lookup(bnext_ref, clo_ref, chi_ref, v):
  u = jnp.minimum(jnp.abs(v), _V_CLIP)
  k = (u * _LUT_INV_STEP).astype(jnp.int32)
  k7 = jnp.bitwise_and(k, 127)
  sel = k < 128
  shape = v.shape[:-1] + (128,)
  def gat(ref):
    t0 = jnp.take_along_axis(jnp.broadcast_to(ref[0:1, :], shape), k7, axis=-1)
    t1 = jnp.take_along_axis(jnp.broadcast_to(ref[1:2, :], shape), k7, axis=-1)
    return jnp.where(sel, t0, t1)
  m = jnp.where(u > gat(bnext_ref), gat(chi_ref), gat(clo_ref))
  return jnp.where(v < 0.0, -m, m)


def _lane64_iota():
  return jnp.bitwise_and(
      jax.lax.broadcasted_iota(jnp.int32, (1, 128), 1), 63)


def _seg_max(m):
  lane64 = _lane64_iota()
  for d in (1, 2, 4, 8, 16, 32):
    r1 = pltpu.roll(m, d, axis=1)
    r2 = pltpu.roll(m, d + 64, axis=1)
    m = jnp.maximum(m, jnp.where(lane64 < d, r2, r1))
  return m


def _quant_kernel(x_ref, a_ref, b_ref, m_ref, bnext_ref, clo_ref, chi_ref,
                  maxcinv_ref, out_ref):
  x = x_ref[...]
  a2 = a_ref[...]
  b2 = b_ref[...]
  msum = m_ref[...]
  max_c = maxcinv_ref[0, 0]

  norms = jnp.sqrt(
      jnp.dot(x * x, msum, preferred_element_type=jnp.float32)) + _EPS
  xu = x / norms
  y = jnp.dot(xu, a2, preferred_element_type=jnp.float32)
  ay = jnp.abs(y)
  maxy = _seg_max(ay)
  rms = maxy / max_c
  recon_u = _lut_lookup(bnext_ref, clo_ref, chi_ref, y / (rms + _EPS))

  num = jnp.dot(y * recon_u, msum, preferred_element_type=jnp.float32)
  den = jnp.dot(recon_u * recon_u, msum,
                preferred_element_type=jnp.float32) + _EPS
  gamma1 = num / den
  recon_2 = _lut_lookup(bnext_ref, clo_ref, chi_ref, y / (gamma1 + _EPS))

  mean_abs = jnp.dot(ay, msum,
                     preferred_element_type=jnp.float32) * (1.0 / _DIM) + _EPS
  is_spiky = (maxy / mean_abs) > 5.0
  gamma = jnp.where(is_spiky, rms, gamma1)
  recon = jnp.where(is_spiky, recon_u, recon_2) * gamma

  x_unit = jnp.dot(recon, b2, preferred_element_type=jnp.float32)
  out_ref[...] = x_unit * norms


@jax.jit
def kernel(x, all_signs, wht_mat, centroids):
  shape = x.shape
  xp = x.astype(jnp.float32).reshape(-1, 2 * _DIM)
  n_packed = xp.shape[0]

  signs = all_signs[0]
  a_mat = (signs[:, None] * wht_mat) * _FINAL_SCALE
  b_mat = (wht_mat * signs[None, :]) * _FINAL_SCALE
  zero = jnp.zeros((_DIM, _DIM), jnp.float32)
  a2 = jnp.concatenate(
      [jnp.concatenate([a_mat, zero], 1), jnp.concatenate([zero, a_mat], 1)], 0)
  b2 = jnp.concatenate(
      [jnp.concatenate([b_mat, zero], 1), jnp.concatenate([zero, b_mat], 1)], 0)
  ones = jnp.ones((_DIM, _DIM), jnp.float32)
  msum = jnp.concatenate(
      [jnp.concatenate([ones, zero], 1), jnp.concatenate([zero, ones], 1)], 0)

  boundaries = (centroids[:-1] + centroids[1:]) * 0.5
  edges = _LUT_LO + _LUT_STEP * jnp.arange(_K_LUT, dtype=jnp.float32)
  base = jnp.sum(boundaries[None, :] < edges[:, None], axis=1,
                 dtype=jnp.int32)
  lev = jnp.arange(_NLEV, dtype=jnp.int32)
  onehot = (base[:, None] == lev[None, :]).astype(jnp.float32)
  bpad = jnp.concatenate([boundaries, boundaries[-1:]])
  bnext = jnp.where(base < _NLEV - 1, jnp.dot(onehot, bpad, precision='highest'), _BIG)
  clo = jnp.dot(onehot, centroids, precision='highest')
  cshift = jnp.concatenate([centroids[1:], centroids[-1:]])
  chi = jnp.dot(onehot, cshift, precision='highest')
  bnext = bnext.reshape(2, 128)
  clo = clo.reshape(2, 128)
  chi = chi.reshape(2, 128)
  maxc_in = centroids[-1].reshape(1, 1)

  rows_blk = min(_PACKED_ROWS_PER_BLOCK, n_packed)
  grid = (n_packed // rows_blk,)

  full = lambda s: pl.BlockSpec(s, lambda i: (0, 0))
  out = pl.pallas_call(
      _quant_kernel,
      grid=grid,
      in_specs=[
          pl.BlockSpec((rows_blk, 2 * _DIM), lambda i: (i, 0)),
          full((2 * _DIM, 2 * _DIM)),
          full((2 * _DIM, 2 * _DIM)),
          full((2 * _DIM, 2 * _DIM)),
          full((2, 128)),
          full((2, 128)),
          full((2, 128)),
          full((1, 1)),
      ],
      out_specs=pl.BlockSpec((rows_blk, 2 * _DIM), lambda i: (i, 0)),
      out_shape=jax.ShapeDtypeStruct((n_packed, 2 * _DIM), jnp.float32),
  )(xp, a2, b2, msum, bnext, clo, chi, maxc_in)
  return out.reshape(shape)


# speedup vs baseline: 1.0385x; 1.0385x over previous
import jax
import jax.numpy as jnp
from jax.experimental import pallas as pl
from jax.experimental.pallas import tpu as pltpu

_DIM = 64
_EPS = 1e-8
_FINAL_SCALE = _DIM ** -0.5
_NLEV = 256

_K_LUT = 256
_LUT_LO = 0.0
_LUT_STEP = 5.12 / _K_LUT
_LUT_INV_STEP = 1.0 / _LUT_STEP
_V_CLIP = 5.1
_BIG = 1e30

_PACKED_ROWS_PER_BLOCK = 4096


def _lut_lookup(bnext_ref, clo_ref, chi_ref, v):
  u = jnp.minimum(jnp.abs(v), _V_CLIP)
  k = (u * _LUT_INV_STEP).astype(jnp.int32)
  k7 = jnp.bitwise_and(k, 127)
  sel = k < 128
  shape = v.shape[:-1] + (128,)
  def gat(ref):
    t0 = jnp.take_along_axis(jnp.broadcast_to(ref[0:1, :], shape), k7, axis=-1)
    t1 = jnp.take_along_axis(jnp.broadcast_to(ref[1:2, :], shape), k7, axis=-1)
    return jnp.where(sel, t0, t1)
  m = jnp.where(u > gat(bnext_ref), gat(chi_ref), gat(clo_ref))
  return jnp.where(v < 0.0, -m, m)


def _lane64_iota():
  return jnp.bitwise_and(
      jax.lax.broadcasted_iota(jnp.int32, (1, 128), 1), 63)


def _seg_max(m):
  lane64 = _lane64_iota()
  for d in (1, 2, 4, 8, 16, 32):
    r1 = pltpu.roll(m, d, axis=1)
    r2 = pltpu.roll(m, d + 64, axis=1)
    m = jnp.maximum(m, jnp.where(lane64 < d, r2, r1))
  return m


def _quant_kernel(x_ref, a_ref, b_ref, m_ref, bnext_ref, clo_ref, chi_ref,
                  maxcinv_ref, out_ref):
  x = x_ref[...]
  a2 = a_ref[...]
  b2 = b_ref[...]
  msum = m_ref[...]
  max_c = maxcinv_ref[0, 0]

  norms = jnp.sqrt(
      jnp.dot(x * x, msum, preferred_element_type=jnp.float32)) + _EPS
  xu = x / norms
  y = jnp.dot(xu, a2, preferred_element_type=jnp.float32)
  ay = jnp.abs(y)
  maxy = _seg_max(ay)
  rms = maxy / max_c
  recon_u = _lut_lookup(bnext_ref, clo_ref, chi_ref, y / (rms + _EPS))

  num = jnp.dot(y * recon_u, msum, preferred_element_type=jnp.float32)
  den = jnp.dot(recon_u * recon_u, msum,
                preferred_element_type=jnp.float32) + _EPS
  gamma1 = num / den
  recon_2 = _lut_lookup(bnext_ref, clo_ref, chi_ref, y / (gamma1 + _EPS))

  mean_abs = jnp.dot(ay, msum,
                     preferred_element_type=jnp.float32) * (1.0 / _DIM) + _EPS
  is_spiky = (maxy / mean_abs) > 5.0
  gamma = jnp.where(is_spiky, rms, gamma1)
  recon = jnp.where(is_spiky, recon_u, recon_2) * gamma

  x_unit = jnp.dot(recon, b2, preferred_element_type=jnp.float32)
  out_ref[...] = x_unit * norms


@jax.jit
def kernel(x, all_signs, wht_mat, centroids):
  shape = x.shape
  xp = x.astype(jnp.float32).reshape(-1, 2 * _DIM)
  n_packed = xp.shape[0]

  signs = all_signs[0]
  a_mat = (signs[:, None] * wht_mat) * _FINAL_SCALE
  b_mat = (wht_mat * signs[None, :]) * _FINAL_SCALE
  zero = jnp.zeros((_DIM, _DIM), jnp.float32)
  a2 = jnp.concatenate(
      [jnp.concatenate([a_mat, zero], 1), jnp.concatenate([zero, a_mat], 1)], 0)
  b2 = jnp.concatenate(
      [jnp.concatenate([b_mat, zero], 1), jnp.concatenate([zero, b_mat], 1)], 0)
  ones = jnp.ones((_DIM, _DIM), jnp.float32)
  msum = jnp.concatenate(
      [jnp.concatenate([ones, zero], 1), jnp.concatenate([zero, ones], 1)], 0)

  boundaries = (centroids[:-1] + centroids[1:]) * 0.5
  edges = _LUT_LO + _LUT_STEP * jnp.arange(_K_LUT, dtype=jnp.float32)
  base = jnp.sum(boundaries[None, :] < edges[:, None], axis=1,
                 dtype=jnp.int32)
  lev = jnp.arange(_NLEV, dtype=jnp.int32)
  onehot = (base[:, None] == lev[None, :]).astype(jnp.float32)
  bpad = jnp.concatenate([boundaries, boundaries[-1:]])
  bnext = jnp.where(base < _NLEV - 1, jnp.dot(onehot, bpad, precision='highest'), _BIG)
  clo = jnp.dot(onehot, centroids, precision='highest')
  cshift = jnp.concatenate([centroids[1:], centroids[-1:]])
  chi = jnp.dot(onehot, cshift, precision='highest')
  bnext = bnext.reshape(2, 128)
  clo = clo.reshape(2, 128)
  chi = chi.reshape(2, 128)
  maxc_in = centroids[-1].reshape(1, 1)

  rows_blk = min(_PACKED_ROWS_PER_BLOCK, n_packed)
  grid = (n_packed // rows_blk,)

  full = lambda s: pl.BlockSpec(s, lambda i: (0, 0))
  out = pl.pallas_call(
      _quant_kernel,
      grid=grid,
      in_specs=[
          pl.BlockSpec((rows_blk, 2 * _DIM), lambda i: (i, 0)),
          full((2 * _DIM, 2 * _DIM)),
          full((2 * _DIM, 2 * _DIM)),
          full((2 * _DIM, 2 * _DIM)),
          full((2, 128)),
          full((2, 128)),
          full((2, 128)),
          full((1, 1)),
      ],
      out_specs=pl.BlockSpec((rows_blk, 2 * _DIM), lambda i: (i, 0)),
      out_shape=jax.ShapeDtypeStruct((n_packed, 2 * _DIM), jnp.float32),
  )(xp, a2, b2, msum, bnext, clo, chi, maxc_in)
  return out.reshape(shape)
